# baseline (speedup 1.0000x reference)
import functools

import jax
import jax.numpy as jnp
from jax import lax
from jax.experimental import pallas as pl
from jax.experimental.pallas import tpu as pltpu

_NQ = 8192
_K = 512
_D = 128
_CS = 2048


def _conv(x, w, stride, pad):
    return lax.conv_general_dilated(
        x, w, (stride, stride), ((pad, pad), (pad, pad)),
        dimension_numbers=('NCHW', 'OIHW', 'NCHW'))


def _conv_t(x, w, stride, pad):
    wp = jnp.flip(jnp.transpose(w, (1, 0, 2, 3)), axis=(2, 3))
    k = w.shape[2]
    p = k - 1 - pad
    return lax.conv_general_dilated(
        x, wp, (1, 1), ((p, p), (p, p)), lhs_dilation=(stride, stride),
        dimension_numbers=('NCHW', 'OIHW', 'NCHW'))


def _res_block(x, w3, w1):
    out = jax.nn.relu(x)
    out = _conv(out, w3, 1, 1)
    out = jax.nn.relu(out)
    out = _conv(out, w1, 1, 0)
    return out + x


def _vq_body(zenc_ref, embd_ref, zdec_ref, tgather_ref):
    embd = embd_ref[...]
    tn = jnp.sum(embd * embd, axis=1)
    colmin = jnp.full((_K,), jnp.inf, jnp.float32)
    colarg = jnp.zeros((_K,), jnp.int32)
    for c in range(_NQ // _CS):
        q = zenc_ref[pl.ds(c * _CS, _CS), :]
        qn = jnp.sum(q * q, axis=1)
        qt = lax.dot_general(q, embd, (((1,), (1,)), ((), ())),
                             preferred_element_type=jnp.float32)
        d2 = qn[:, None] + tn[None, :] - 2.0 * qt
        rowmin = jnp.min(d2, axis=1)
        jcol = lax.broadcasted_iota(jnp.int32, (_CS, _K), 1)
        ridx = jnp.min(jnp.where(d2 == rowmin[:, None], jcol, _K), axis=1)
        onehot = (ridx[:, None] == jcol).astype(jnp.float32)
        zdec_ref[pl.ds(c * _CS, _CS), :] = lax.dot_general(
            onehot, embd, (((1,), (0,)), ((), ())),
            preferred_element_type=jnp.float32,
            precision=lax.Precision.HIGHEST)
        cmin = jnp.min(d2, axis=0)
        irow = lax.broadcasted_iota(jnp.int32, (_CS, _K), 0)
        carg = jnp.min(jnp.where(d2 == cmin[None, :], irow, _CS), axis=0) + c * _CS
        upd = cmin < colmin
        colarg = jnp.where(upd, carg, colarg)
        colmin = jnp.where(upd, cmin, colmin)
    acc = jnp.zeros((_K, _D), jnp.float32)
    for c in range(_NQ // _CS):
        ilocal = lax.broadcasted_iota(jnp.int32, (_K, _CS), 1) + c * _CS
        oh = (colarg[:, None] == ilocal).astype(jnp.float32)
        acc = acc + lax.dot_general(
            oh, zenc_ref[pl.ds(c * _CS, _CS), :], (((1,), (0,)), ((), ())),
            preferred_element_type=jnp.float32,
            precision=lax.Precision.HIGHEST)
    tgather_ref[...] = acc


@functools.partial(jax.jit, static_argnums=())
def _vq(zenc, embd):
    return pl.pallas_call(
        _vq_body,
        out_shape=[
            jax.ShapeDtypeStruct((_NQ, _D), jnp.float32),
            jax.ShapeDtypeStruct((_K, _D), jnp.float32),
        ],
    )(zenc, embd)


def kernel(x, enc_c1, enc_c2, enc_r1w1, enc_r1w2, enc_r2w1, enc_r2w2, embd,
           dec_r1w1, dec_r1w2, dec_r2w1, dec_r2w2, dec_t1, dec_t2):
    z = _conv(x, enc_c1, 2, 1)
    z = _conv(z, enc_c2, 2, 1)
    z = _res_block(z, enc_r1w1, enc_r1w2)
    Z_enc_ori = _res_block(z, enc_r2w1, enc_r2w2)
    z_bs, z_c, z_w, z_h = Z_enc_ori.shape
    Z_enc = jnp.transpose(Z_enc_ori, (0, 2, 3, 1)).reshape(-1, _D)
    Z_dec_flat, Z_enc_for_embd = _vq(Z_enc, embd)
    Z_dec = jnp.transpose(Z_dec_flat.reshape(z_bs, z_w, z_h, z_c), (0, 3, 1, 2))
    y = _res_block(Z_dec, dec_r1w1, dec_r1w2)
    y = _res_block(y, dec_r2w1, dec_r2w2)
    y = _conv_t(y, dec_t1, 2, 1)
    y = _conv_t(y, dec_t2, 2, 1)
    X_recon = jnp.tanh(y)
    return (X_recon, Z_enc_ori, Z_dec, Z_enc_for_embd)
